# baseline (speedup 1.0000x reference)
import functools

import jax
import jax.numpy as jnp
from jax.experimental import pallas as pl
from jax.experimental.pallas import tpu as pltpu

POOL = 30
TOPK = 5
LEN = 5
D = 768
B = 128
S = 197
PROMPT_ROWS = TOPK * LEN
OUT_S = PROMPT_ROWS + S


CB = 16
NCHUNK = B // CB
ALIGNED_S = 216


def _copy_mean_body(x_ref, out_ref, mean_ref, inbuf, outbuf,
                    sem_in, sem_out_a, sem_out_b):
    def in_copy(i):
        return pltpu.make_async_copy(
            x_ref.at[pl.ds(i * CB, CB)], inbuf.at[i % 2], sem_in.at[i % 2])

    def out_copy_a(i):
        return pltpu.make_async_copy(
            outbuf.at[i % 2, slice(None), pl.ds(0, ALIGNED_S)],
            out_ref.at[pl.ds(i * CB, CB), pl.ds(0, ALIGNED_S)],
            sem_out_a.at[i % 2])

    def out_copy_b(i):
        return pltpu.make_async_copy(
            outbuf.at[i % 2, slice(None), pl.ds(ALIGNED_S, OUT_S - ALIGNED_S)],
            out_ref.at[pl.ds(i * CB, CB), pl.ds(ALIGNED_S, OUT_S - ALIGNED_S)],
            sem_out_b.at[i % 2])

    in_copy(0).start()
    for i in range(NCHUNK):
        if i >= 2:
            out_copy_a(i - 2).wait()
            out_copy_b(i - 2).wait()
        if i + 1 < NCHUNK:
            in_copy(i + 1).start()
        in_copy(i).wait()
        x = inbuf[i % 2]
        mean_ref[pl.ds(i * CB, CB), 0, :] = jnp.sum(x, axis=1) / jnp.float32(S)
        outbuf[i % 2, :, PROMPT_ROWS:OUT_S, :] = x
        out_copy_a(i).start()
        out_copy_b(i).start()
    out_copy_a(NCHUNK - 2).wait()
    out_copy_b(NCHUNK - 2).wait()
    out_copy_a(NCHUNK - 1).wait()
    out_copy_b(NCHUNK - 1).wait()


def _route_body(mean_ref, pk_ref, prompt_ref, xhead_ref, big_in_ref,
                big_out_ref, sim_ref, idx_ref, rs_ref):
    del big_in_ref
    xm = mean_ref[:, 0, :]
    pk = pk_ref[:]
    pkn = pk * jax.lax.rsqrt(
        jnp.maximum(jnp.sum(pk * pk, axis=1, keepdims=True), jnp.float32(1e-12)))
    xn = xm * jax.lax.rsqrt(
        jnp.maximum(jnp.sum(xm * xm, axis=1, keepdims=True), jnp.float32(1e-12)))
    sim = jnp.dot(xn, pkn.T, preferred_element_type=jnp.float32)
    sim_ref[:] = sim

    iota_pool = jax.lax.broadcasted_iota(jnp.int32, (B, POOL), 1)
    masked = sim
    picked = jnp.zeros((B, POOL), jnp.float32)
    for _ in range(TOPK):
        rowmax = jnp.max(masked, axis=1, keepdims=True)
        am = jnp.min(jnp.where(masked == rowmax, iota_pool, POOL), axis=1)
        onehot = (iota_pool == am[:, None]).astype(jnp.float32)
        picked = picked + onehot
        masked = jnp.where(onehot > 0, -jnp.inf, masked)
    counts = jnp.sum(picked, axis=0, keepdims=True)

    iota_row = jax.lax.broadcasted_iota(jnp.int32, (1, POOL), 1)
    iota_k = jax.lax.broadcasted_iota(jnp.int32, (1, TOPK), 1)
    cm = counts
    majors = jnp.zeros((1, TOPK), jnp.int32)
    iota_col = jax.lax.broadcasted_iota(jnp.int32, (POOL, 1), 0)
    sel_mask = jnp.zeros((POOL, 1), jnp.float32)
    for k in range(TOPK):
        cmax = jnp.max(cm, axis=1, keepdims=True)
        mk = jnp.min(jnp.where(cm == cmax, iota_row, POOL), axis=1)
        onehot_m = (iota_row == mk[:, None]).astype(jnp.float32)
        sel_mask = sel_mask + (iota_col == mk[:, None]).astype(jnp.float32)
        majors = jnp.where(iota_k == k, mk[:, None], majors)
        cm = jnp.where(onehot_m > 0, jnp.float32(-1.0), cm)
    idx_ref[:] = jnp.broadcast_to(majors, (B, TOPK))

    comb = jnp.sum(pkn * sel_mask, axis=0, keepdims=True)
    s1 = jnp.sum(xn * comb, axis=0, keepdims=True)
    rs_ref[:, :] = jnp.sum(s1, axis=1, keepdims=True) / jnp.float32(B)

    for k in range(TOPK):
        mk_s = majors[0, k]
        sub = prompt_ref[pl.ds(mk_s, 1), :, :]
        big_out_ref[:, k * LEN:(k + 1) * LEN, :] = jnp.broadcast_to(
            sub, (B, LEN, D))
    big_out_ref[:, PROMPT_ROWS:32, :] = xhead_ref[:, 0:32 - PROMPT_ROWS, :]


@functools.partial(jax.jit)
def kernel(x_embed, prompt, prompt_key):
    big, mean = pl.pallas_call(
        _copy_mean_body,
        grid=(1,),
        compiler_params=pltpu.CompilerParams(
            vmem_limit_bytes=100 * 1024 * 1024,
        ),
        in_specs=[pl.BlockSpec(memory_space=pl.ANY)],
        out_specs=[
            pl.BlockSpec(memory_space=pl.ANY),
            pl.BlockSpec((B, 1, D), lambda i: (0, 0, 0)),
        ],
        out_shape=[
            jax.ShapeDtypeStruct((B, OUT_S, D), jnp.float32),
            jax.ShapeDtypeStruct((B, 1, D), jnp.float32),
        ],
        scratch_shapes=[
            pltpu.VMEM((2, CB, S, D), jnp.float32),
            pltpu.VMEM((2, CB, OUT_S, D), jnp.float32),
            pltpu.SemaphoreType.DMA((2,)),
            pltpu.SemaphoreType.DMA((2,)),
            pltpu.SemaphoreType.DMA((2,)),
        ],
    )(x_embed)

    big2, sim, idx, rs = pl.pallas_call(
        _route_body,
        grid=(1,),
        in_specs=[
            pl.BlockSpec((B, 1, D), lambda i: (0, 0, 0)),
            pl.BlockSpec((POOL, D), lambda i: (0, 0)),
            pl.BlockSpec((POOL, LEN, D), lambda i: (0, 0, 0)),
            pl.BlockSpec((B, 8, D), lambda i: (0, 0, 0)),
            pl.BlockSpec(memory_space=pl.ANY),
        ],
        out_specs=[
            pl.BlockSpec((B, 32, D), lambda i: (0, 0, 0)),
            pl.BlockSpec((B, POOL), lambda i: (0, 0)),
            pl.BlockSpec((B, TOPK), lambda i: (0, 0)),
            pl.BlockSpec((1, 1), lambda i: (0, 0)),
        ],
        out_shape=[
            jax.ShapeDtypeStruct((B, OUT_S, D), jnp.float32),
            jax.ShapeDtypeStruct((B, POOL), jnp.float32),
            jax.ShapeDtypeStruct((B, TOPK), jnp.int32),
            jax.ShapeDtypeStruct((1, 1), jnp.float32),
        ],
        input_output_aliases={4: 0},
    )(mean, prompt_key, prompt, x_embed, big)

    return big2, rs[0, 0], sim, idx


# speedup vs baseline: 1.0336x; 1.0336x over previous
import functools

import jax
import jax.numpy as jnp
from jax.experimental import pallas as pl
from jax.experimental.pallas import tpu as pltpu

POOL = 30
TOPK = 5
LEN = 5
D = 768
B = 128
S = 197
PROMPT_ROWS = TOPK * LEN
OUT_S = PROMPT_ROWS + S

CB = 16
NCHUNK = B // CB
FILL_S = 32
LOOP_LO = FILL_S
LOOP_HI = 216


def _fused_body(x_ref, pk_ref, prompt_ref, out_ref, sim_ref, idx_ref, rs_ref,
                inbuf, outbuf, mean_scr, head_scr, fill_scr,
                sem_in, sem_out_a, sem_out_b, sem_fill):
    def in_copy(i):
        return pltpu.make_async_copy(
            x_ref.at[pl.ds(i * CB, CB)], inbuf.at[i % 2], sem_in.at[i % 2])

    def out_copy_a(i):
        return pltpu.make_async_copy(
            outbuf.at[i % 2, slice(None), pl.ds(LOOP_LO, LOOP_HI - LOOP_LO)],
            out_ref.at[pl.ds(i * CB, CB), pl.ds(LOOP_LO, LOOP_HI - LOOP_LO)],
            sem_out_a.at[i % 2])

    def out_copy_b(i):
        return pltpu.make_async_copy(
            outbuf.at[i % 2, slice(None), pl.ds(LOOP_HI, OUT_S - LOOP_HI)],
            out_ref.at[pl.ds(i * CB, CB), pl.ds(LOOP_HI, OUT_S - LOOP_HI)],
            sem_out_b.at[i % 2])

    in_copy(0).start()
    for i in range(NCHUNK):
        if i >= 2:
            out_copy_a(i - 2).wait()
            out_copy_b(i - 2).wait()
        if i + 1 < NCHUNK:
            in_copy(i + 1).start()
        in_copy(i).wait()
        x = inbuf[i % 2]
        mean_scr[pl.ds(i * CB, CB), :] = jnp.sum(x, axis=1) / jnp.float32(S)
        head_scr[pl.ds(i * CB, CB), :, :] = x[:, 0:FILL_S - PROMPT_ROWS, :]
        outbuf[i % 2, :, PROMPT_ROWS:OUT_S, :] = x
        out_copy_a(i).start()
        out_copy_b(i).start()

    xm = mean_scr[:, :]
    pk = pk_ref[:]
    pkn = pk * jax.lax.rsqrt(
        jnp.maximum(jnp.sum(pk * pk, axis=1, keepdims=True), jnp.float32(1e-12)))
    xn = xm * jax.lax.rsqrt(
        jnp.maximum(jnp.sum(xm * xm, axis=1, keepdims=True), jnp.float32(1e-12)))
    sim = jnp.dot(xn, pkn.T, preferred_element_type=jnp.float32)
    sim_ref[:] = sim

    iota_pool = jax.lax.broadcasted_iota(jnp.int32, (B, POOL), 1)
    masked = sim
    picked = jnp.zeros((B, POOL), jnp.float32)
    for _ in range(TOPK):
        rowmax = jnp.max(masked, axis=1, keepdims=True)
        am = jnp.min(jnp.where(masked == rowmax, iota_pool, POOL), axis=1)
        onehot = (iota_pool == am[:, None]).astype(jnp.float32)
        picked = picked + onehot
        masked = jnp.where(onehot > 0, -jnp.inf, masked)
    counts = jnp.sum(picked, axis=0, keepdims=True)

    iota_row = jax.lax.broadcasted_iota(jnp.int32, (1, POOL), 1)
    iota_k = jax.lax.broadcasted_iota(jnp.int32, (1, TOPK), 1)
    iota_col = jax.lax.broadcasted_iota(jnp.int32, (POOL, 1), 0)
    cm = counts
    majors = jnp.zeros((1, TOPK), jnp.int32)
    sel_mask = jnp.zeros((POOL, 1), jnp.float32)
    for k in range(TOPK):
        cmax = jnp.max(cm, axis=1, keepdims=True)
        mk = jnp.min(jnp.where(cm == cmax, iota_row, POOL), axis=1)
        onehot_m = (iota_row == mk[:, None]).astype(jnp.float32)
        sel_mask = sel_mask + (iota_col == mk[:, None]).astype(jnp.float32)
        majors = jnp.where(iota_k == k, mk[:, None], majors)
        cm = jnp.where(onehot_m > 0, jnp.float32(-1.0), cm)
    idx_ref[:] = jnp.broadcast_to(majors, (B, TOPK))

    comb = jnp.sum(pkn * sel_mask, axis=0, keepdims=True)
    s1 = jnp.sum(xn * comb, axis=0, keepdims=True)
    rs_ref[:, :] = jnp.sum(s1, axis=1, keepdims=True) / jnp.float32(B)

    for k in range(TOPK):
        mk_s = majors[0, k]
        sub = prompt_ref[pl.ds(mk_s, 1), :, :]
        fill_scr[:, k * LEN:(k + 1) * LEN, :] = jnp.broadcast_to(
            sub, (B, LEN, D))
    fill_scr[:, PROMPT_ROWS:FILL_S, :] = head_scr[:, :, :]

    fills = [
        pltpu.make_async_copy(
            fill_scr.at[pl.ds(c * CB, CB)],
            out_ref.at[pl.ds(c * CB, CB), pl.ds(0, FILL_S)],
            sem_fill)
        for c in range(NCHUNK)
    ]
    for f in fills:
        f.start()
    out_copy_a(NCHUNK - 2).wait()
    out_copy_b(NCHUNK - 2).wait()
    out_copy_a(NCHUNK - 1).wait()
    out_copy_b(NCHUNK - 1).wait()
    for f in fills:
        f.wait()


@functools.partial(jax.jit)
def kernel(x_embed, prompt, prompt_key):
    big, sim, idx, rs = pl.pallas_call(
        _fused_body,
        grid=(1,),
        compiler_params=pltpu.CompilerParams(
            vmem_limit_bytes=110 * 1024 * 1024,
        ),
        in_specs=[
            pl.BlockSpec(memory_space=pl.ANY),
            pl.BlockSpec((POOL, D), lambda i: (0, 0)),
            pl.BlockSpec((POOL, LEN, D), lambda i: (0, 0, 0)),
        ],
        out_specs=[
            pl.BlockSpec(memory_space=pl.ANY),
            pl.BlockSpec((B, POOL), lambda i: (0, 0)),
            pl.BlockSpec((B, TOPK), lambda i: (0, 0)),
            pl.BlockSpec((1, 1), lambda i: (0, 0)),
        ],
        out_shape=[
            jax.ShapeDtypeStruct((B, OUT_S, D), jnp.float32),
            jax.ShapeDtypeStruct((B, POOL), jnp.float32),
            jax.ShapeDtypeStruct((B, TOPK), jnp.int32),
            jax.ShapeDtypeStruct((1, 1), jnp.float32),
        ],
        scratch_shapes=[
            pltpu.VMEM((2, CB, S, D), jnp.float32),
            pltpu.VMEM((2, CB, OUT_S, D), jnp.float32),
            pltpu.VMEM((B, D), jnp.float32),
            pltpu.VMEM((B, FILL_S - PROMPT_ROWS, D), jnp.float32),
            pltpu.VMEM((B, FILL_S, D), jnp.float32),
            pltpu.SemaphoreType.DMA((2,)),
            pltpu.SemaphoreType.DMA((2,)),
            pltpu.SemaphoreType.DMA((2,)),
            pltpu.SemaphoreType.DMA,
        ],
    )(x_embed, prompt_key, prompt)

    return big, rs[0, 0], sim, idx


# speedup vs baseline: 1.0382x; 1.0045x over previous
import functools

import jax
import jax.numpy as jnp
from jax.experimental import pallas as pl
from jax.experimental.pallas import tpu as pltpu

POOL = 30
TOPK = 5
LEN = 5
D = 768
B = 128
S = 197
PROMPT_ROWS = TOPK * LEN
OUT_S = PROMPT_ROWS + S

CB = 16
NCHUNK = B // CB
FILL_S = 32
LOOP_LO = FILL_S
LOOP_HI = 216


def _fused_body(x_ref, pk_ref, prompt_ref, out_ref, sim_ref, idx_ref, rs_ref,
                inbuf, outbuf, mean_scr, head_scr, tail_scr, fill_scr,
                sem_in, sem_out_a, sem_out_b, sem_fill):
    def in_copy(i):
        return pltpu.make_async_copy(
            x_ref.at[pl.ds(i * CB, CB)], inbuf.at[i % 2], sem_in.at[i % 2])

    def out_copy_a(i):
        return pltpu.make_async_copy(
            outbuf.at[i % 2, slice(None), pl.ds(LOOP_LO, LOOP_HI - LOOP_LO)],
            out_ref.at[pl.ds(i * CB, CB), pl.ds(LOOP_LO, LOOP_HI - LOOP_LO)],
            sem_out_a.at[i % 2])

    tail_dma = pltpu.make_async_copy(
        tail_scr, out_ref.at[slice(None), pl.ds(LOOP_HI, OUT_S - LOOP_HI)],
        sem_out_b)

    in_copy(0).start()
    for i in range(NCHUNK):
        if i >= 2:
            out_copy_a(i - 2).wait()
        if i + 1 < NCHUNK:
            in_copy(i + 1).start()
        in_copy(i).wait()
        x = inbuf[i % 2]
        mean_scr[pl.ds(i * CB, CB), :] = jnp.sum(x, axis=1) / jnp.float32(S)
        head_scr[pl.ds(i * CB, CB), :, :] = x[:, 0:FILL_S - PROMPT_ROWS, :]
        tail_scr[pl.ds(i * CB, CB), :, :] = x[:, LOOP_HI - PROMPT_ROWS:S, :]
        outbuf[i % 2, :, PROMPT_ROWS:OUT_S, :] = x
        out_copy_a(i).start()
    tail_dma.start()

    xm = mean_scr[:, :]
    pk = pk_ref[:]
    pkn = pk * jax.lax.rsqrt(
        jnp.maximum(jnp.sum(pk * pk, axis=1, keepdims=True), jnp.float32(1e-12)))
    xn = xm * jax.lax.rsqrt(
        jnp.maximum(jnp.sum(xm * xm, axis=1, keepdims=True), jnp.float32(1e-12)))
    sim = jnp.dot(xn, pkn.T, preferred_element_type=jnp.float32)
    sim_ref[:] = sim

    iota_pool = jax.lax.broadcasted_iota(jnp.int32, (B, POOL), 1)
    masked = sim
    picked = jnp.zeros((B, POOL), jnp.float32)
    for _ in range(TOPK):
        rowmax = jnp.max(masked, axis=1, keepdims=True)
        am = jnp.min(jnp.where(masked == rowmax, iota_pool, POOL), axis=1)
        onehot = (iota_pool == am[:, None]).astype(jnp.float32)
        picked = picked + onehot
        masked = jnp.where(onehot > 0, -jnp.inf, masked)
    counts = jnp.sum(picked, axis=0, keepdims=True)

    iota_row = jax.lax.broadcasted_iota(jnp.int32, (1, POOL), 1)
    iota_k = jax.lax.broadcasted_iota(jnp.int32, (1, TOPK), 1)
    iota_col = jax.lax.broadcasted_iota(jnp.int32, (POOL, 1), 0)
    cm = counts
    majors = jnp.zeros((1, TOPK), jnp.int32)
    sel_mask = jnp.zeros((POOL, 1), jnp.float32)
    for k in range(TOPK):
        cmax = jnp.max(cm, axis=1, keepdims=True)
        mk = jnp.min(jnp.where(cm == cmax, iota_row, POOL), axis=1)
        onehot_m = (iota_row == mk[:, None]).astype(jnp.float32)
        sel_mask = sel_mask + (iota_col == mk[:, None]).astype(jnp.float32)
        majors = jnp.where(iota_k == k, mk[:, None], majors)
        cm = jnp.where(onehot_m > 0, jnp.float32(-1.0), cm)
    idx_ref[:] = jnp.broadcast_to(majors, (B, TOPK))

    comb = jnp.sum(pkn * sel_mask, axis=0, keepdims=True)
    s1 = jnp.sum(xn * comb, axis=0, keepdims=True)
    rs_ref[:, :] = jnp.sum(s1, axis=1, keepdims=True) / jnp.float32(B)

    for k in range(TOPK):
        mk_s = majors[0, k]
        sub = prompt_ref[pl.ds(mk_s, 1), :, :]
        fill_scr[:, k * LEN:(k + 1) * LEN, :] = jnp.broadcast_to(
            sub, (B, LEN, D))
    fill_scr[:, PROMPT_ROWS:FILL_S, :] = head_scr[:, :, :]

    fill_dma = pltpu.make_async_copy(
        fill_scr, out_ref.at[slice(None), pl.ds(0, FILL_S)], sem_fill)
    fill_dma.start()
    out_copy_a(NCHUNK - 2).wait()
    out_copy_a(NCHUNK - 1).wait()
    tail_dma.wait()
    fill_dma.wait()


@functools.partial(jax.jit)
def kernel(x_embed, prompt, prompt_key):
    big, sim, idx, rs = pl.pallas_call(
        _fused_body,
        grid=(1,),
        compiler_params=pltpu.CompilerParams(
            vmem_limit_bytes=110 * 1024 * 1024,
        ),
        in_specs=[
            pl.BlockSpec(memory_space=pl.ANY),
            pl.BlockSpec((POOL, D), lambda i: (0, 0)),
            pl.BlockSpec((POOL, LEN, D), lambda i: (0, 0, 0)),
        ],
        out_specs=[
            pl.BlockSpec(memory_space=pl.ANY),
            pl.BlockSpec((B, POOL), lambda i: (0, 0)),
            pl.BlockSpec((B, TOPK), lambda i: (0, 0)),
            pl.BlockSpec((1, 1), lambda i: (0, 0)),
        ],
        out_shape=[
            jax.ShapeDtypeStruct((B, OUT_S, D), jnp.float32),
            jax.ShapeDtypeStruct((B, POOL), jnp.float32),
            jax.ShapeDtypeStruct((B, TOPK), jnp.int32),
            jax.ShapeDtypeStruct((1, 1), jnp.float32),
        ],
        scratch_shapes=[
            pltpu.VMEM((2, CB, S, D), jnp.float32),
            pltpu.VMEM((2, CB, OUT_S, D), jnp.float32),
            pltpu.VMEM((B, D), jnp.float32),
            pltpu.VMEM((B, FILL_S - PROMPT_ROWS, D), jnp.float32),
            pltpu.VMEM((B, OUT_S - LOOP_HI, D), jnp.float32),
            pltpu.VMEM((B, FILL_S, D), jnp.float32),
            pltpu.SemaphoreType.DMA((2,)),
            pltpu.SemaphoreType.DMA((2,)),
            pltpu.SemaphoreType.DMA,
            pltpu.SemaphoreType.DMA,
        ],
    )(x_embed, prompt_key, prompt)

    return big, rs[0, 0], sim, idx
